# baseline (speedup 1.0000x reference)
import jax
import jax.numpy as jnp
from jax.experimental import pallas as pl
from jax.experimental.pallas import tpu as pltpu

T, H, L, V, B, MAXN = 8192, 512, 128, 1024, 256, 15
R = 256
NB = T // R


def _mm(a, b):
    return jnp.dot(a.astype(jnp.bfloat16), b, preferred_element_type=jnp.float32)


def _body(idx_ref, ctx_ref, hnei_ref, tv_ref, emb_ref,
          wz_ref, wr_ref, ur_ref, wh_ref, w1_ref, w2_ref, wo_ref,
          ui_ref, u1_ref, u2_ref, bias_ref,
          word_ref, stop_ref,
          x_s, r1_s, sumh_s, gated_s):
    f32 = jnp.float32
    bf16 = jnp.bfloat16
    n = pl.program_id(1)

    h = hnei_ref[...].reshape(R, H)
    h_bf = h.astype(bf16)

    @pl.when(n == 0)
    def _init():
        idx = idx_ref[0, 0, :]
        iota_v = jax.lax.broadcasted_iota(jnp.int32, (R, V), 1)
        oh_x = (idx[:, None] == iota_v).astype(bf16)
        x = jnp.dot(oh_x, emb_ref[...], preferred_element_type=f32)
        x_s[...] = x
        r1_s[...] = _mm(x, wr_ref[...]) + bias_ref[1, :H][None, :]
        sumh_s[...] = jnp.zeros((R, H), f32)
        gated_s[...] = jnp.zeros((R, H), f32)

    g = jax.nn.sigmoid(r1_s[...] + jnp.dot(h_bf, ur_ref[...],
                                           preferred_element_type=f32))
    sumh_s[...] += h
    gated_s[...] += g * h

    @pl.when(n == MAXN - 1)
    def _finish():
        x = x_s[...]
        sum_h = sumh_s[...]
        sum_gated = gated_s[...]

        wz_b = bias_ref[0, :H]
        wh_b = bias_ref[2, :H]
        w_b = bias_ref[3, :H]
        ui_b = bias_ref[4, :H]
        u_b = bias_ref[5, :H]
        uo_row = bias_ref[6, :H]
        uo_b = bias_ref[7, 0]

        ctx = ctx_ref[0, 0, :]
        iota_b = jax.lax.broadcasted_iota(jnp.int32, (R, B), 1)
        oh_c = (ctx[:, None] == iota_b).astype(bf16)
        tc = jnp.dot(oh_c, tv_ref[...], preferred_element_type=f32)

        z = jax.nn.sigmoid(_mm(x, wz_ref[:H, :]) + _mm(sum_h, wz_ref[H:, :])
                           + wz_b[None, :])
        pre_h = jnp.tanh(_mm(x, wh_ref[:H, :]) + _mm(sum_gated, wh_ref[H:, :])
                         + wh_b[None, :])
        new_h = (1.0 - z) * sum_h + z * pre_h

        wh_act = jax.nn.relu(_mm(new_h, w1_ref[...]) + _mm(tc, w2_ref[...])
                             + w_b[None, :])
        word = _mm(wh_act, wo_ref[...])
        word_ref[...] = word + bias_ref[8:8 + (V // H), :].reshape(1, V)

        sh = jax.nn.relu(_mm(x, ui_ref[:H, :]) + _mm(sum_h, ui_ref[H:, :])
                         + ui_b[None, :])
        sh2 = jax.nn.relu(_mm(sh, u1_ref[...]) + _mm(tc, u2_ref[...])
                          + u_b[None, :])
        stop = jnp.sum(sh2 * uo_row[None, :], axis=1, keepdims=True) + uo_b
        stop_ref[...] = jnp.broadcast_to(stop, (R, 128))


@jax.jit
def _run(cur_x_idx, contexts, cur_h_nei, tree_vecs, emb, Wz_w, Wz_b, Wr_w,
         Wr_b, Ur_w, Wh_w, Wh_b, W_w, W_b, Wo_w, Wo_b, Ui_w, Ui_b, U_w, U_b,
         Uo_w, Uo_b):
    f32 = jnp.float32
    bf16 = jnp.bfloat16
    idx2 = cur_x_idx.astype(jnp.int32).reshape(NB, 1, R)
    ctx2 = contexts.astype(jnp.int32).reshape(NB, 1, R)

    wz = Wz_w.T.astype(bf16)
    wr = Wr_w.T.astype(bf16)
    ur = Ur_w.T.astype(bf16)
    wh = Wh_w.T.astype(bf16)
    w1 = W_w.T[:H, :].astype(bf16)
    w2 = W_w.T[H:, :].astype(bf16)
    wo = Wo_w.T.astype(bf16)
    ui = Ui_w.T.astype(bf16)
    u1 = U_w.T[:H, :].astype(bf16)
    u2 = U_w.T[H:, :].astype(bf16)
    emb_bf = emb.astype(bf16)
    tv_bf = tree_vecs.astype(bf16)

    bias = jnp.stack([
        Wz_b, Wr_b, Wh_b, W_b, Ui_b, U_b, Uo_w[0, :],
        jnp.full((H,), Uo_b[0], f32),
    ], axis=0)
    bias = jnp.concatenate([bias, Wo_b.reshape(V // H, H)], axis=0)

    full = lambda shape: pl.BlockSpec(shape, lambda i, n: (0,) * len(shape))
    grid = (NB, MAXN)
    in_specs = [
            pl.BlockSpec((1, 1, R), lambda i, n: (i, 0, 0)),
            pl.BlockSpec((1, 1, R), lambda i, n: (i, 0, 0)),
            pl.BlockSpec((R, 1, 1, H), lambda i, n: (i, n, 0, 0)),
            full((B, L)),
            full((V, H)),
            full((2 * H, H)),
            full((H, H)),
            full((H, H)),
            full((2 * H, H)),
            full((H, H)),
            full((L, H)),
            full((H, V)),
            full((2 * H, H)),
            full((H, H)),
            full((L, H)),
            full((8 + V // H, H)),
    ]
    out_specs = [
        pl.BlockSpec((R, V), lambda i, n: (i, 0)),
        pl.BlockSpec((R, 128), lambda i, n: (i, 0)),
    ]
    word, stop = pl.pallas_call(
        _body,
        grid=grid,
        in_specs=in_specs,
        out_specs=out_specs,
        out_shape=[
            jax.ShapeDtypeStruct((T, V), f32),
            jax.ShapeDtypeStruct((T, 128), f32),
        ],
        scratch_shapes=[
            pltpu.VMEM((R, H), f32),
            pltpu.VMEM((R, H), f32),
            pltpu.VMEM((R, H), f32),
            pltpu.VMEM((R, H), f32),
        ],
    )(idx2, ctx2, cur_h_nei.reshape(T, MAXN, 1, H), tv_bf, emb_bf, wz, wr, ur, wh, w1, w2, wo,
      ui, u1, u2, bias)
    return jnp.concatenate([word, stop[:, :1]], axis=1)


def kernel(cur_x_idx, contexts, cur_h_nei, tree_vecs, emb, Wz_w, Wz_b, Wr_w,
           Wr_b, Ur_w, Wh_w, Wh_b, W_w, W_b, Wo_w, Wo_b, Ui_w, Ui_b, U_w,
           U_b, Uo_w, Uo_b):
    return _run(cur_x_idx, contexts, cur_h_nei, tree_vecs, emb, Wz_w, Wz_b,
                Wr_w, Wr_b, Ur_w, Wh_w, Wh_b, W_w, W_b, Wo_w, Wo_b, Ui_w,
                Ui_b, U_w, U_b, Uo_w, Uo_b)


# speedup vs baseline: 1.3270x; 1.3270x over previous
import jax
import jax.numpy as jnp
from jax.experimental import pallas as pl

T, H, L, V, B, MAXN = 8192, 512, 128, 1024, 256, 15
R = 128
RN = R * MAXN
NB = T // R


def _mm(a, b):
    return jnp.dot(a.astype(jnp.bfloat16), b, preferred_element_type=jnp.float32)


def _body(idx_ref, ctx_ref, hnei_ref, s_ref, st_ref, tv_ref, emb_ref,
          wz_ref, wr_ref, ur_ref, wh_ref, w1_ref, w2_ref, wo_ref,
          ui_ref, u1_ref, u2_ref, bias_ref,
          word_ref, stop_ref):
    f32 = jnp.float32
    bf16 = jnp.bfloat16
    idx = idx_ref[0, 0, :]
    ctx = ctx_ref[0, 0, :]

    x2 = hnei_ref[...]
    x2b = x2.astype(bf16)
    s_mat = s_ref[...]
    st_mat = st_ref[...]

    sum_h = jnp.dot(s_mat, x2b, preferred_element_type=f32)

    iota_v = jax.lax.broadcasted_iota(jnp.int32, (R, V), 1)
    oh_x = (idx[:, None] == iota_v).astype(bf16)
    x = jnp.dot(oh_x, emb_ref[...], preferred_element_type=f32)

    iota_b = jax.lax.broadcasted_iota(jnp.int32, (R, B), 1)
    oh_c = (ctx[:, None] == iota_b).astype(bf16)
    tc = jnp.dot(oh_c, tv_ref[...], preferred_element_type=f32)

    wz_b = bias_ref[0, :H]
    wr_b = bias_ref[1, :H]
    wh_b = bias_ref[2, :H]
    w_b = bias_ref[3, :H]
    ui_b = bias_ref[4, :H]
    u_b = bias_ref[5, :H]
    uo_row = bias_ref[6, :H]
    uo_b = bias_ref[7, 0]

    r1 = _mm(x, wr_ref[...]) + wr_b[None, :]
    r1_full = _mm(st_mat, r1.astype(bf16))
    r2 = jnp.dot(x2b, ur_ref[...], preferred_element_type=f32)
    g = jax.nn.sigmoid(r1_full + r2)
    p = (g * x2).astype(bf16)
    sum_gated = jnp.dot(s_mat, p, preferred_element_type=f32)

    z = jax.nn.sigmoid(_mm(x, wz_ref[:H, :]) + _mm(sum_h, wz_ref[H:, :])
                       + wz_b[None, :])
    pre_h = jnp.tanh(_mm(x, wh_ref[:H, :]) + _mm(sum_gated, wh_ref[H:, :])
                     + wh_b[None, :])
    new_h = (1.0 - z) * sum_h + z * pre_h

    wh_act = jax.nn.relu(_mm(new_h, w1_ref[...]) + _mm(tc, w2_ref[...])
                         + w_b[None, :])
    word = _mm(wh_act, wo_ref[...])
    word_ref[...] = word + bias_ref[8:8 + (V // H), :].reshape(1, V)

    sh = jax.nn.relu(_mm(x, ui_ref[:H, :]) + _mm(sum_h, ui_ref[H:, :])
                     + ui_b[None, :])
    sh2 = jax.nn.relu(_mm(sh, u1_ref[...]) + _mm(tc, u2_ref[...])
                      + u_b[None, :])
    stop = jnp.sum(sh2 * uo_row[None, :], axis=1, keepdims=True) + uo_b
    stop_ref[...] = jnp.broadcast_to(stop, (R, 128))


@jax.jit
def _run(cur_x_idx, contexts, cur_h_nei, tree_vecs, emb, Wz_w, Wz_b, Wr_w,
         Wr_b, Ur_w, Wh_w, Wh_b, W_w, W_b, Wo_w, Wo_b, Ui_w, Ui_b, U_w, U_b,
         Uo_w, Uo_b):
    f32 = jnp.float32
    bf16 = jnp.bfloat16
    idx2 = cur_x_idx.astype(jnp.int32).reshape(NB, 1, R)
    ctx2 = contexts.astype(jnp.int32).reshape(NB, 1, R)
    hnei2 = cur_h_nei.reshape(T * MAXN, H)

    seg = (jnp.arange(R)[:, None] ==
           (jnp.arange(RN) // MAXN)[None, :]).astype(bf16)
    seg_t = seg.T

    wz = Wz_w.T.astype(bf16)
    wr = Wr_w.T.astype(bf16)
    ur = Ur_w.T.astype(bf16)
    wh = Wh_w.T.astype(bf16)
    w1 = W_w.T[:H, :].astype(bf16)
    w2 = W_w.T[H:, :].astype(bf16)
    wo = Wo_w.T.astype(bf16)
    ui = Ui_w.T.astype(bf16)
    u1 = U_w.T[:H, :].astype(bf16)
    u2 = U_w.T[H:, :].astype(bf16)
    emb_bf = emb.astype(bf16)
    tv_bf = tree_vecs.astype(bf16)

    bias = jnp.stack([
        Wz_b, Wr_b, Wh_b, W_b, Ui_b, U_b, Uo_w[0, :],
        jnp.full((H,), Uo_b[0], f32),
    ], axis=0)
    bias = jnp.concatenate([bias, Wo_b.reshape(V // H, H)], axis=0)

    full = lambda shape: pl.BlockSpec(shape, lambda i: (0,) * len(shape))
    grid = (NB,)
    in_specs = [
        pl.BlockSpec((1, 1, R), lambda i: (i, 0, 0)),
        pl.BlockSpec((1, 1, R), lambda i: (i, 0, 0)),
        pl.BlockSpec((RN, H), lambda i: (i, 0)),
        full((R, RN)),
        full((RN, R)),
        full((B, L)),
        full((V, H)),
        full((2 * H, H)),
        full((H, H)),
        full((H, H)),
        full((2 * H, H)),
        full((H, H)),
        full((L, H)),
        full((H, V)),
        full((2 * H, H)),
        full((H, H)),
        full((L, H)),
        full((8 + V // H, H)),
    ]
    out_specs = [
        pl.BlockSpec((R, V), lambda i: (i, 0)),
        pl.BlockSpec((R, 128), lambda i: (i, 0)),
    ]
    word, stop = pl.pallas_call(
        _body,
        grid=grid,
        in_specs=in_specs,
        out_specs=out_specs,
        out_shape=[
            jax.ShapeDtypeStruct((T, V), f32),
            jax.ShapeDtypeStruct((T, 128), f32),
        ],
    )(idx2, ctx2, hnei2, seg, seg_t, tv_bf, emb_bf, wz, wr, ur, wh, w1, w2,
      wo, ui, u1, u2, bias)
    return jnp.concatenate([word, stop[:, :1]], axis=1)


def kernel(cur_x_idx, contexts, cur_h_nei, tree_vecs, emb, Wz_w, Wz_b, Wr_w,
           Wr_b, Ur_w, Wh_w, Wh_b, W_w, W_b, Wo_w, Wo_b, Ui_w, Ui_b, U_w,
           U_b, Uo_w, Uo_b):
    return _run(cur_x_idx, contexts, cur_h_nei, tree_vecs, emb, Wz_w, Wz_b,
                Wr_w, Wr_b, Ur_w, Wh_w, Wh_b, W_w, W_b, Wo_w, Wo_b, Ui_w,
                Ui_b, U_w, U_b, Uo_w, Uo_b)
